# baseline (speedup 1.0000x reference)
import functools

import jax
import jax.numpy as jnp
from jax import lax
from jax.experimental import pallas as pl
from jax.experimental.pallas import tpu as pltpu
from jax.experimental.pallas import tpu_sc as plsc

N_NODES = 10000
N_EDGES = 320000
BN_EPS = 1e-5

_NC = 2
_NS = 16
_NW = _NC * _NS
_L = 16
_NPAD = 10240


def _deg_body(col_hbm, out_hbm, colbuf, degbuf):
    wid = lax.axis_index("s") * _NC + lax.axis_index("c")
    pltpu.sync_copy(col_hbm.at[pl.ds(wid * _EPW, _EPW)], colbuf)

    zeros16 = jnp.zeros((_L,), jnp.float32)
    ones16 = jnp.ones((_L,), jnp.float32)

    def _zero(i, _):
        degbuf[pl.ds(i * _L, _L)] = zeros16
        return ()

    lax.fori_loop(0, _NPAD // _L, _zero, (), unroll=8)

    def _hist(i, _):
        idx = colbuf[pl.ds(i * _L, _L)]
        plsc.addupdate_scatter(degbuf, [idx], ones16)
        return ()

    lax.fori_loop(0, _EPW // _L, _hist, (), unroll=8)
    pltpu.sync_copy(degbuf, out_hbm.at[wid])


@functools.partial(jax.jit, static_argnames=())
def _deg_partials(col):
    mesh = plsc.VectorSubcoreMesh(core_axis_name="c", subcore_axis_name="s")
    return pl.kernel(
        _deg_body,
        out_type=jax.ShapeDtypeStruct((_NW, _NPAD), jnp.float32),
        mesh=mesh,
        scratch_types=[
            pltpu.VMEM((_EPW,), jnp.int32),
            pltpu.VMEM((_NPAD,), jnp.float32),
        ],
        compiler_params=pltpu.CompilerParams(needs_layout_passes=False),
    )(col)


_CHUNK = 125
_NCHUNK = 80
_HCHUNK = _NCHUNK // 2
_EPW = _NCHUNK * _CHUNK
_ZROWS = 128
_RPT = _NPAD // _NS


def _make_agg(D, tc_tiling=True):
    def body(g_hbm, row_hbm, col_hbm, out_hbm, rowbuf, colbuf, msg, acc,
             gsem0, gsem1):
        c = lax.axis_index("c")
        s = lax.axis_index("s")
        wid = s * _NC + c

        zeros16 = jnp.zeros((_L,), jnp.float32)

        def _zero(i, _):
            msg[i // (D // _L), pl.ds((i % (D // _L)) * _L, _L)] = zeros16
            return ()

        lax.fori_loop(0, _ZROWS * D // _L, _zero, (), unroll=8)

        for j in range(_RPT // _ZROWS):
            pltpu.sync_copy(msg.at[pl.ds(0, _ZROWS)],
                            acc.at[pl.ds(s * _RPT + j * _ZROWS, _ZROWS)])
        plsc.subcore_barrier()

        slot0 = msg.at[pl.ds(0, _CHUNK)]
        slot1 = msg.at[pl.ds(_CHUNK, _CHUNK)]

        for half in range(2):
            pltpu.sync_copy(row_hbm.at[wid, half], rowbuf)
            pltpu.sync_copy(col_hbm.at[wid, half], colbuf)

            pltpu.async_copy(g_hbm.at[rowbuf.at[0]], slot0, gsem0)

            def _step(i, _):
                k0 = 2 * i
                pltpu.async_copy(g_hbm.at[rowbuf.at[k0 + 1]], slot1, gsem1)
                pltpu.make_async_copy(g_hbm.at[rowbuf.at[k0]], slot0,
                                      gsem0).wait()
                pltpu.sync_copy(slot0, acc.at[colbuf.at[k0]], add=True)

                @pl.when(k0 + 2 < _HCHUNK)
                def _():
                    pltpu.async_copy(g_hbm.at[rowbuf.at[k0 + 2]], slot0, gsem0)

                pltpu.make_async_copy(g_hbm.at[rowbuf.at[k0 + 1]], slot1,
                                      gsem1).wait()
                pltpu.sync_copy(slot1, acc.at[colbuf.at[k0 + 1]], add=True)
                return ()

            lax.fori_loop(0, _HCHUNK // 2, _step, ())
        plsc.subcore_barrier()
        pltpu.sync_copy(acc.at[pl.ds(s * _RPT, _RPT)],
                        out_hbm.at[c, pl.ds(s * _RPT, _RPT)])

    mesh = plsc.VectorSubcoreMesh(core_axis_name="c", subcore_axis_name="s")
    return pl.kernel(
        body,
        out_type=jax.ShapeDtypeStruct((_NC, _NPAD, D), jnp.float32),
        mesh=mesh,
        scratch_types=[
            pltpu.VMEM((_HCHUNK, _CHUNK), jnp.int32),
            pltpu.VMEM((_HCHUNK, _CHUNK), jnp.int32),
            pltpu.VMEM((2 * _CHUNK, D), jnp.float32),
            pltpu.VMEM_SHARED((_NPAD, D), jnp.float32),
            pltpu.SemaphoreType.DMA,
            pltpu.SemaphoreType.DMA,
        ],
        compiler_params=pltpu.CompilerParams(
            needs_layout_passes=False, use_tc_tiling_on_sc=tc_tiling),
    )


_B = 1000
_NB = N_NODES // _B


_DB = 1024


def _mm1_body(xp_ref, w1_ref, h1_ref):
    h1_ref[...] = jnp.dot(xp_ref[...], w1_ref[...],
                          preferred_element_type=jnp.float32)


def _mm1(xp, W1):
    return pl.pallas_call(
        _mm1_body,
        grid=(_NPAD // _DB,),
        in_specs=[
            pl.BlockSpec((_DB, 128), lambda j: (j, 0)),
            pl.BlockSpec((128, 128), lambda j: (0, 0)),
        ],
        out_specs=pl.BlockSpec((_DB, 128), lambda j: (j, 0)),
        out_shape=jax.ShapeDtypeStruct((_NPAD, 128), jnp.float32),
    )(xp, W1)


def _prep_body(h1_ref, degp_ref, g1_ref, disb_ref):
    ones = jnp.ones((_NW, 1), jnp.float32)
    deg = 1.0 + lax.dot_general(degp_ref[...], ones, (((0,), (0,)), ((), ())),
                                preferred_element_type=jnp.float32)
    dis = lax.rsqrt(deg)
    disb_ref[...] = jnp.broadcast_to(dis, (_DB, 128))
    g1_ref[...] = h1_ref[...] * dis


def _prep(h1, degp):
    return pl.pallas_call(
        _prep_body,
        grid=(_NPAD // _DB,),
        in_specs=[
            pl.BlockSpec((_DB, 128), lambda j: (j, 0)),
            pl.BlockSpec((_NW, _DB), lambda j: (0, j)),
        ],
        out_specs=[
            pl.BlockSpec((_DB, 128), lambda j: (j, 0)),
            pl.BlockSpec((_DB, 128), lambda j: (j, 0)),
        ],
        out_shape=[
            jax.ShapeDtypeStruct((_NPAD, 128), jnp.float32),
            jax.ShapeDtypeStruct((_NPAD, 128), jnp.float32),
        ],
    )(h1, degp)


def _bn_body(aggp_ref, g1_ref, disb_ref, b1_ref, gam_ref, bet_ref, w2_ref,
             g2_ref, t_scr, stat_scr):
    p = pl.program_id(0)
    j = pl.program_id(1)

    @pl.when(p == 0)
    def _pass1():
        t = disb_ref[...] * (aggp_ref[0] + aggp_ref[1] + g1_ref[...]) \
            + b1_ref[...]
        t_scr[pl.ds(j * _B, _B), :] = t

        @pl.when(j == 0)
        def _init():
            stat_scr[...] = jnp.zeros_like(stat_scr)

        stat_scr[0:1, :] += jnp.sum(t, axis=0, keepdims=True)
        stat_scr[1:2, :] += jnp.sum(t * t, axis=0, keepdims=True)

    @pl.when(p == 1)
    def _pass2():
        t = t_scr[pl.ds(j * _B, _B), :]
        mean = stat_scr[0:1, :] * (1.0 / N_NODES)
        var = stat_scr[1:2, :] * (1.0 / N_NODES) - mean * mean
        hn = gam_ref[...] * (t - mean) * lax.rsqrt(var + BN_EPS) + bet_ref[...]
        h = jnp.maximum(hn, 0.0)
        g2_ref[...] = jnp.dot(h, w2_ref[...],
                              preferred_element_type=jnp.float32) \
            * disb_ref[:, 0:1]


def _bn_mm2(aggp, g1, disb, b1r, gammar, betar, W2p):
    return pl.pallas_call(
        _bn_body,
        grid=(2, _NB),
        in_specs=[
            pl.BlockSpec((_NC, _B, 128), lambda p, j: (0, j, 0)),
            pl.BlockSpec((_B, 128), lambda p, j: (j, 0)),
            pl.BlockSpec((_B, 128), lambda p, j: (j, 0)),
            pl.BlockSpec((1, 128), lambda p, j: (0, 0)),
            pl.BlockSpec((1, 128), lambda p, j: (0, 0)),
            pl.BlockSpec((1, 128), lambda p, j: (0, 0)),
            pl.BlockSpec((128, 48), lambda p, j: (0, 0)),
        ],
        out_specs=pl.BlockSpec((_B, 48), lambda p, j: (j, 0)),
        out_shape=jax.ShapeDtypeStruct((N_NODES, 48), jnp.float32),
        scratch_shapes=[
            pltpu.VMEM((N_NODES, 128), jnp.float32),
            pltpu.VMEM((8, 128), jnp.float32),
        ],
    )(aggp, g1, disb, b1r, gammar, betar, W2p)


def _final_body(aggp_ref, g2_ref, disb_ref, b2_ref, out_ref):
    a = aggp_ref[0, :, :40] + aggp_ref[1, :, :40] + g2_ref[:, :40]
    out_ref[...] = disb_ref[:, 0:40] * a + b2_ref[...]


def _final(aggp, g2, disb, b2r):
    return pl.pallas_call(
        _final_body,
        grid=(_NB,),
        in_specs=[
            pl.BlockSpec((_NC, _B, 48), lambda j: (0, j, 0)),
            pl.BlockSpec((_B, 48), lambda j: (j, 0)),
            pl.BlockSpec((_B, 128), lambda j: (j, 0)),
            pl.BlockSpec((1, 40), lambda j: (0, 0)),
        ],
        out_specs=pl.BlockSpec((_B, 40), lambda j: (j, 0)),
        out_shape=jax.ShapeDtypeStruct((N_NODES, 40), jnp.float32),
    )(aggp, g2, disb, b2r)


def kernel(x, edge_index, W1, b1, gamma, beta, W2, b2):
    row = edge_index[0]
    col = edge_index[1]
    row3d = row.reshape(_NW, 2, _HCHUNK, _CHUNK)
    col3d = col.reshape(_NW, 2, _HCHUNK, _CHUNK)

    degp = _deg_partials(col)
    xp = jnp.pad(x, ((0, _NPAD - N_NODES), (0, 0)))
    h1 = _mm1(xp, W1)

    g1, disb = _prep(h1, degp)
    agg1p = _make_agg(128)(g1, row3d, col3d)

    W2p = jnp.pad(W2, ((0, 0), (0, 8)))
    g2 = _bn_mm2(agg1p, g1, disb, b1.reshape(1, 128),
                 gamma.reshape(1, 128), beta.reshape(1, 128), W2p)
    agg2p = _make_agg(48, tc_tiling=False)(g2, row3d, col3d)

    return _final(agg2p, g2, disb, b2.reshape(1, 40))


# speedup vs baseline: 1.0433x; 1.0433x over previous
import functools

import jax
import jax.numpy as jnp
from jax import lax
from jax.experimental import pallas as pl
from jax.experimental.pallas import tpu as pltpu
from jax.experimental.pallas import tpu_sc as plsc

N_NODES = 10000
N_EDGES = 320000
BN_EPS = 1e-5

_NC = 2
_NS = 16
_NW = _NC * _NS
_L = 16
_NPAD = 10240


def _deg_body(col_hbm, out_hbm, colbuf, degbuf):
    wid = lax.axis_index("s") * _NC + lax.axis_index("c")
    pltpu.sync_copy(col_hbm.at[pl.ds(wid * _EPW, _EPW)], colbuf)

    zeros16 = jnp.zeros((_L,), jnp.float32)
    ones16 = jnp.ones((_L,), jnp.float32)

    def _zero(i, _):
        degbuf[pl.ds(i * _L, _L)] = zeros16
        return ()

    lax.fori_loop(0, _NPAD // _L, _zero, (), unroll=8)

    def _hist(i, _):
        idx = colbuf[pl.ds(i * _L, _L)]
        plsc.addupdate_scatter(degbuf, [idx], ones16)
        return ()

    lax.fori_loop(0, _EPW // _L, _hist, (), unroll=8)
    pltpu.sync_copy(degbuf, out_hbm.at[wid])


@functools.partial(jax.jit, static_argnames=())
def _deg_partials(col):
    mesh = plsc.VectorSubcoreMesh(core_axis_name="c", subcore_axis_name="s")
    return pl.kernel(
        _deg_body,
        out_type=jax.ShapeDtypeStruct((_NW, _NPAD), jnp.float32),
        mesh=mesh,
        scratch_types=[
            pltpu.VMEM((_EPW,), jnp.int32),
            pltpu.VMEM((_NPAD,), jnp.float32),
        ],
        compiler_params=pltpu.CompilerParams(needs_layout_passes=False),
    )(col)


_CHUNK = 125
_NCHUNK = 80
_HCHUNK = _NCHUNK // 2
_EPW = _NCHUNK * _CHUNK
_ZROWS = 128
_RPT = _NPAD // _NS


def _make_agg(D, tc_tiling=True):
    def body(g_hbm, row_hbm, col_hbm, out_hbm, rowbuf, colbuf, msg, acc,
             gsem0, gsem1):
        c = lax.axis_index("c")
        s = lax.axis_index("s")
        wid = s * _NC + c

        zeros16 = jnp.zeros((_L,), jnp.float32)

        def _zero(i, _):
            msg[i // (D // _L), pl.ds((i % (D // _L)) * _L, _L)] = zeros16
            return ()

        lax.fori_loop(0, _ZROWS * D // _L, _zero, (), unroll=8)

        for j in range(_RPT // _ZROWS):
            pltpu.sync_copy(msg.at[pl.ds(0, _ZROWS)],
                            acc.at[pl.ds(s * _RPT + j * _ZROWS, _ZROWS)])
        plsc.subcore_barrier()

        slot0 = msg.at[pl.ds(0, _CHUNK)]
        slot1 = msg.at[pl.ds(_CHUNK, _CHUNK)]

        for half in range(2):
            pltpu.sync_copy(row_hbm.at[wid, half], rowbuf)
            pltpu.sync_copy(col_hbm.at[wid, half], colbuf)

            pltpu.async_copy(g_hbm.at[rowbuf.at[0]], slot0, gsem0)

            def _step(i, _):
                k0 = 2 * i
                pltpu.async_copy(g_hbm.at[rowbuf.at[k0 + 1]], slot1, gsem1)
                pltpu.make_async_copy(g_hbm.at[rowbuf.at[k0]], slot0,
                                      gsem0).wait()
                pltpu.sync_copy(slot0, acc.at[colbuf.at[k0]], add=True)

                @pl.when(k0 + 2 < _HCHUNK)
                def _():
                    pltpu.async_copy(g_hbm.at[rowbuf.at[k0 + 2]], slot0, gsem0)

                pltpu.make_async_copy(g_hbm.at[rowbuf.at[k0 + 1]], slot1,
                                      gsem1).wait()
                pltpu.sync_copy(slot1, acc.at[colbuf.at[k0 + 1]], add=True)
                return ()

            lax.fori_loop(0, _HCHUNK // 2, _step, ())
        plsc.subcore_barrier()
        pltpu.sync_copy(acc.at[pl.ds(s * _RPT, _RPT)],
                        out_hbm.at[c, pl.ds(s * _RPT, _RPT)])

    mesh = plsc.VectorSubcoreMesh(core_axis_name="c", subcore_axis_name="s")
    return pl.kernel(
        body,
        out_type=jax.ShapeDtypeStruct((_NC, _NPAD, D), jnp.float32),
        mesh=mesh,
        scratch_types=[
            pltpu.VMEM((_HCHUNK, _CHUNK), jnp.int32),
            pltpu.VMEM((_HCHUNK, _CHUNK), jnp.int32),
            pltpu.VMEM((2 * _CHUNK, D), jnp.float32),
            pltpu.VMEM_SHARED((_NPAD, D), jnp.float32),
            pltpu.SemaphoreType.DMA,
            pltpu.SemaphoreType.DMA,
        ],
        compiler_params=pltpu.CompilerParams(
            needs_layout_passes=False, use_tc_tiling_on_sc=tc_tiling),
    )


_B = 1000
_NB = N_NODES // _B


_DB = 1024


def _prep_body(xp_ref, w1_ref, degp_ref, g1_ref, disb_ref):
    ones = jnp.ones((_NW, 1), jnp.float32)
    deg = 1.0 + lax.dot_general(degp_ref[...], ones, (((0,), (0,)), ((), ())),
                                preferred_element_type=jnp.float32)
    dis = lax.rsqrt(deg)
    disb_ref[...] = jnp.broadcast_to(dis, (_DB, 128))
    h = jnp.dot(xp_ref[...], w1_ref[...], preferred_element_type=jnp.float32)
    g1_ref[...] = h * dis


def _prep(xp, W1, degp):
    return pl.pallas_call(
        _prep_body,
        grid=(_NPAD // _DB,),
        in_specs=[
            pl.BlockSpec((_DB, 128), lambda j: (j, 0)),
            pl.BlockSpec((128, 128), lambda j: (0, 0)),
            pl.BlockSpec((_NW, _DB), lambda j: (0, j)),
        ],
        out_specs=[
            pl.BlockSpec((_DB, 128), lambda j: (j, 0)),
            pl.BlockSpec((_DB, 128), lambda j: (j, 0)),
        ],
        out_shape=[
            jax.ShapeDtypeStruct((_NPAD, 128), jnp.float32),
            jax.ShapeDtypeStruct((_NPAD, 128), jnp.float32),
        ],
    )(xp, W1, degp)


def _bn_body(aggp_ref, g1_ref, disb_ref, b1_ref, gam_ref, bet_ref, w2_ref,
             g2_ref, t_scr, stat_scr):
    p = pl.program_id(0)
    j = pl.program_id(1)

    @pl.when(p == 0)
    def _pass1():
        t = disb_ref[...] * (aggp_ref[0] + aggp_ref[1] + g1_ref[...]) \
            + b1_ref[...]
        t_scr[pl.ds(j * _B, _B), :] = t

        @pl.when(j == 0)
        def _init():
            stat_scr[...] = jnp.zeros_like(stat_scr)

        stat_scr[0:1, :] += jnp.sum(t, axis=0, keepdims=True)
        stat_scr[1:2, :] += jnp.sum(t * t, axis=0, keepdims=True)

    @pl.when(p == 1)
    def _pass2():
        t = t_scr[pl.ds(j * _B, _B), :]
        mean = stat_scr[0:1, :] * (1.0 / N_NODES)
        var = stat_scr[1:2, :] * (1.0 / N_NODES) - mean * mean
        hn = gam_ref[...] * (t - mean) * lax.rsqrt(var + BN_EPS) + bet_ref[...]
        h = jnp.maximum(hn, 0.0)
        g2_ref[...] = jnp.dot(h, w2_ref[...],
                              preferred_element_type=jnp.float32) \
            * disb_ref[:, 0:1]


def _bn_mm2(aggp, g1, disb, b1r, gammar, betar, W2p):
    return pl.pallas_call(
        _bn_body,
        grid=(2, _NB),
        in_specs=[
            pl.BlockSpec((_NC, _B, 128), lambda p, j: (0, j * (1 - p), 0)),
            pl.BlockSpec((_B, 128), lambda p, j: (j * (1 - p), 0)),
            pl.BlockSpec((_B, 128), lambda p, j: (j, 0)),
            pl.BlockSpec((1, 128), lambda p, j: (0, 0)),
            pl.BlockSpec((1, 128), lambda p, j: (0, 0)),
            pl.BlockSpec((1, 128), lambda p, j: (0, 0)),
            pl.BlockSpec((128, 48), lambda p, j: (0, 0)),
        ],
        out_specs=pl.BlockSpec((_B, 48), lambda p, j: (j, 0)),
        out_shape=jax.ShapeDtypeStruct((N_NODES, 48), jnp.float32),
        scratch_shapes=[
            pltpu.VMEM((N_NODES, 128), jnp.float32),
            pltpu.VMEM((8, 128), jnp.float32),
        ],
    )(aggp, g1, disb, b1r, gammar, betar, W2p)


def _final_body(aggp_ref, g2_ref, disb_ref, b2_ref, out_ref):
    a = aggp_ref[0, :, :40] + aggp_ref[1, :, :40] + g2_ref[:, :40]
    out_ref[...] = disb_ref[:, 0:40] * a + b2_ref[...]


def _final(aggp, g2, disb, b2r):
    return pl.pallas_call(
        _final_body,
        grid=(_NB,),
        in_specs=[
            pl.BlockSpec((_NC, _B, 48), lambda j: (0, j, 0)),
            pl.BlockSpec((_B, 48), lambda j: (j, 0)),
            pl.BlockSpec((_B, 128), lambda j: (j, 0)),
            pl.BlockSpec((1, 40), lambda j: (0, 0)),
        ],
        out_specs=pl.BlockSpec((_B, 40), lambda j: (j, 0)),
        out_shape=jax.ShapeDtypeStruct((N_NODES, 40), jnp.float32),
    )(aggp, g2, disb, b2r)


def kernel(x, edge_index, W1, b1, gamma, beta, W2, b2):
    row = edge_index[0]
    col = edge_index[1]
    row3d = row.reshape(_NW, 2, _HCHUNK, _CHUNK)
    col3d = col.reshape(_NW, 2, _HCHUNK, _CHUNK)

    degp = _deg_partials(col)

    xp = jnp.pad(x, ((0, _NPAD - N_NODES), (0, 0)))
    g1, disb = _prep(xp, W1, degp)
    agg1p = _make_agg(128)(g1, row3d, col3d)

    W2p = jnp.pad(W2, ((0, 0), (0, 8)))
    g2 = _bn_mm2(agg1p, g1, disb, b1.reshape(1, 128),
                 gamma.reshape(1, 128), beta.reshape(1, 128), W2p)
    agg2p = _make_agg(48, tc_tiling=False)(g2, row3d, col3d)

    return _final(agg2p, g2, disb, b2.reshape(1, 40))


# speedup vs baseline: 1.0486x; 1.0051x over previous
import functools

import jax
import jax.numpy as jnp
from jax import lax
from jax.experimental import pallas as pl
from jax.experimental.pallas import tpu as pltpu
from jax.experimental.pallas import tpu_sc as plsc

N_NODES = 10000
N_EDGES = 320000
BN_EPS = 1e-5

_NC = 2
_NS = 16
_NW = _NC * _NS
_L = 16
_NPAD = 10240


def _deg_body(col_hbm, out_hbm, colbuf, degbuf):
    wid = lax.axis_index("s") * _NC + lax.axis_index("c")
    pltpu.sync_copy(col_hbm.at[pl.ds(wid * _EPW, _EPW)], colbuf)

    zeros16 = jnp.zeros((_L,), jnp.float32)
    ones16 = jnp.ones((_L,), jnp.float32)

    def _zero(i, _):
        degbuf[pl.ds(i * _L, _L)] = zeros16
        return ()

    lax.fori_loop(0, _NPAD // _L, _zero, (), unroll=8)

    def _hist(i, _):
        idx = colbuf[pl.ds(i * _L, _L)]
        plsc.addupdate_scatter(degbuf, [idx], ones16)
        return ()

    lax.fori_loop(0, _EPW // _L, _hist, (), unroll=8)
    pltpu.sync_copy(degbuf, out_hbm.at[wid])


@functools.partial(jax.jit, static_argnames=())
def _deg_partials(col):
    mesh = plsc.VectorSubcoreMesh(core_axis_name="c", subcore_axis_name="s")
    return pl.kernel(
        _deg_body,
        out_type=jax.ShapeDtypeStruct((_NW, _NPAD), jnp.float32),
        mesh=mesh,
        scratch_types=[
            pltpu.VMEM((_EPW,), jnp.int32),
            pltpu.VMEM((_NPAD,), jnp.float32),
        ],
        compiler_params=pltpu.CompilerParams(needs_layout_passes=False),
    )(col)


_CHUNK = 125
_NCHUNK = 80
_HCHUNK = _NCHUNK // 2
_EPW = _NCHUNK * _CHUNK
_ZROWS = 128
_RPT = _NPAD // _NS


def _make_agg(D, tc_tiling=True, halves=2):
    spc = _NCHUNK // halves

    def body(g_hbm, row_hbm, col_hbm, out_hbm, rowbuf, colbuf, msg, acc,
             gsem0, gsem1):
        c = lax.axis_index("c")
        s = lax.axis_index("s")
        wid = s * _NC + c

        zeros16 = jnp.zeros((_L,), jnp.float32)

        def _zero(i, _):
            msg[i // (D // _L), pl.ds((i % (D // _L)) * _L, _L)] = zeros16
            return ()

        lax.fori_loop(0, _ZROWS * D // _L, _zero, (), unroll=8)

        for j in range(_RPT // _ZROWS):
            pltpu.sync_copy(msg.at[pl.ds(0, _ZROWS)],
                            acc.at[pl.ds(s * _RPT + j * _ZROWS, _ZROWS)])
        plsc.subcore_barrier()

        slot0 = msg.at[pl.ds(0, _CHUNK)]
        slot1 = msg.at[pl.ds(_CHUNK, _CHUNK)]

        for half in range(halves):
            pltpu.sync_copy(row_hbm.at[wid, half], rowbuf)
            pltpu.sync_copy(col_hbm.at[wid, half], colbuf)

            pltpu.async_copy(g_hbm.at[rowbuf.at[0]], slot0, gsem0)

            def _step(i, _):
                k0 = 2 * i
                pltpu.async_copy(g_hbm.at[rowbuf.at[k0 + 1]], slot1, gsem1)
                pltpu.make_async_copy(g_hbm.at[rowbuf.at[k0]], slot0,
                                      gsem0).wait()
                pltpu.sync_copy(slot0, acc.at[colbuf.at[k0]], add=True)

                @pl.when(k0 + 2 < spc)
                def _():
                    pltpu.async_copy(g_hbm.at[rowbuf.at[k0 + 2]], slot0, gsem0)

                pltpu.make_async_copy(g_hbm.at[rowbuf.at[k0 + 1]], slot1,
                                      gsem1).wait()
                pltpu.sync_copy(slot1, acc.at[colbuf.at[k0 + 1]], add=True)
                return ()

            lax.fori_loop(0, spc // 2, _step, ())
        plsc.subcore_barrier()
        pltpu.sync_copy(acc.at[pl.ds(s * _RPT, _RPT)],
                        out_hbm.at[c, pl.ds(s * _RPT, _RPT)])

    mesh = plsc.VectorSubcoreMesh(core_axis_name="c", subcore_axis_name="s")
    return pl.kernel(
        body,
        out_type=jax.ShapeDtypeStruct((_NC, _NPAD, D), jnp.float32),
        mesh=mesh,
        scratch_types=[
            pltpu.VMEM((spc, _CHUNK), jnp.int32),
            pltpu.VMEM((spc, _CHUNK), jnp.int32),
            pltpu.VMEM((2 * _CHUNK, D), jnp.float32),
            pltpu.VMEM_SHARED((_NPAD, D), jnp.float32),
            pltpu.SemaphoreType.DMA,
            pltpu.SemaphoreType.DMA,
        ],
        compiler_params=pltpu.CompilerParams(
            needs_layout_passes=False, use_tc_tiling_on_sc=tc_tiling),
    )


_B = 1000
_NB = N_NODES // _B


_DB = 1024


def _prep_body(xp_ref, w1_ref, degp_ref, g1_ref, disb_ref):
    ones = jnp.ones((_NW, 1), jnp.float32)
    deg = 1.0 + lax.dot_general(degp_ref[...], ones, (((0,), (0,)), ((), ())),
                                preferred_element_type=jnp.float32)
    dis = lax.rsqrt(deg)
    disb_ref[...] = jnp.broadcast_to(dis, (_DB, 128))
    h = jnp.dot(xp_ref[...], w1_ref[...], preferred_element_type=jnp.float32)
    g1_ref[...] = h * dis


def _prep(xp, W1, degp):
    return pl.pallas_call(
        _prep_body,
        grid=(_NPAD // _DB,),
        in_specs=[
            pl.BlockSpec((_DB, 128), lambda j: (j, 0)),
            pl.BlockSpec((128, 128), lambda j: (0, 0)),
            pl.BlockSpec((_NW, _DB), lambda j: (0, j)),
        ],
        out_specs=[
            pl.BlockSpec((_DB, 128), lambda j: (j, 0)),
            pl.BlockSpec((_DB, 128), lambda j: (j, 0)),
        ],
        out_shape=[
            jax.ShapeDtypeStruct((_NPAD, 128), jnp.float32),
            jax.ShapeDtypeStruct((_NPAD, 128), jnp.float32),
        ],
    )(xp, W1, degp)


def _bn_body(aggp_ref, g1_ref, disb_ref, b1_ref, gam_ref, bet_ref, w2_ref,
             g2_ref, t_scr, stat_scr):
    p = pl.program_id(0)
    j = pl.program_id(1)

    @pl.when(p == 0)
    def _pass1():
        t = disb_ref[...] * (aggp_ref[0] + aggp_ref[1] + g1_ref[...]) \
            + b1_ref[...]
        t_scr[pl.ds(j * _B, _B), :] = t

        @pl.when(j == 0)
        def _init():
            stat_scr[...] = jnp.zeros_like(stat_scr)

        stat_scr[0:1, :] += jnp.sum(t, axis=0, keepdims=True)
        stat_scr[1:2, :] += jnp.sum(t * t, axis=0, keepdims=True)

    @pl.when(p == 1)
    def _pass2():
        t = t_scr[pl.ds(j * _B, _B), :]
        mean = stat_scr[0:1, :] * (1.0 / N_NODES)
        var = stat_scr[1:2, :] * (1.0 / N_NODES) - mean * mean
        hn = gam_ref[...] * (t - mean) * lax.rsqrt(var + BN_EPS) + bet_ref[...]
        h = jnp.maximum(hn, 0.0)
        g2_ref[...] = jnp.dot(h, w2_ref[...],
                              preferred_element_type=jnp.float32) \
            * disb_ref[:, 0:1]


def _bn_mm2(aggp, g1, disb, b1r, gammar, betar, W2p):
    return pl.pallas_call(
        _bn_body,
        grid=(2, _NB),
        in_specs=[
            pl.BlockSpec((_NC, _B, 128), lambda p, j: (0, j * (1 - p), 0)),
            pl.BlockSpec((_B, 128), lambda p, j: (j * (1 - p), 0)),
            pl.BlockSpec((_B, 128), lambda p, j: (j, 0)),
            pl.BlockSpec((1, 128), lambda p, j: (0, 0)),
            pl.BlockSpec((1, 128), lambda p, j: (0, 0)),
            pl.BlockSpec((1, 128), lambda p, j: (0, 0)),
            pl.BlockSpec((128, 48), lambda p, j: (0, 0)),
        ],
        out_specs=pl.BlockSpec((_B, 48), lambda p, j: (j, 0)),
        out_shape=jax.ShapeDtypeStruct((N_NODES, 48), jnp.float32),
        scratch_shapes=[
            pltpu.VMEM((N_NODES, 128), jnp.float32),
            pltpu.VMEM((8, 128), jnp.float32),
        ],
    )(aggp, g1, disb, b1r, gammar, betar, W2p)


def _final_body(aggp_ref, g2_ref, disb_ref, b2_ref, out_ref):
    a = aggp_ref[0, :, :40] + aggp_ref[1, :, :40] + g2_ref[:, :40]
    out_ref[...] = disb_ref[:, 0:40] * a + b2_ref[...]


def _final(aggp, g2, disb, b2r):
    return pl.pallas_call(
        _final_body,
        grid=(_NB,),
        in_specs=[
            pl.BlockSpec((_NC, _B, 48), lambda j: (0, j, 0)),
            pl.BlockSpec((_B, 48), lambda j: (j, 0)),
            pl.BlockSpec((_B, 128), lambda j: (j, 0)),
            pl.BlockSpec((1, 40), lambda j: (0, 0)),
        ],
        out_specs=pl.BlockSpec((_B, 40), lambda j: (j, 0)),
        out_shape=jax.ShapeDtypeStruct((N_NODES, 40), jnp.float32),
    )(aggp, g2, disb, b2r)


def kernel(x, edge_index, W1, b1, gamma, beta, W2, b2):
    row = edge_index[0]
    col = edge_index[1]
    row3d = row.reshape(_NW, 2, _HCHUNK, _CHUNK)
    col3d = col.reshape(_NW, 2, _HCHUNK, _CHUNK)

    degp = _deg_partials(col)

    xp = jnp.pad(x, ((0, _NPAD - N_NODES), (0, 0)))
    g1, disb = _prep(xp, W1, degp)
    agg1p = _make_agg(128)(g1, row3d, col3d)

    W2p = jnp.pad(W2, ((0, 0), (0, 8)))
    g2 = _bn_mm2(agg1p, g1, disb, b1.reshape(1, 128),
                 gamma.reshape(1, 128), beta.reshape(1, 128), W2p)
    row3d1 = row.reshape(_NW, 1, _NCHUNK, _CHUNK)
    col3d1 = col.reshape(_NW, 1, _NCHUNK, _CHUNK)
    agg2p = _make_agg(48, tc_tiling=False, halves=1)(g2, row3d1, col3d1)

    return _final(agg2p, g2, disb, b2.reshape(1, 40))
